# baseline (device time: 128774 ns/iter reference)
import jax
import jax.numpy as jnp
from jax import lax
from jax.experimental import pallas as pl
from jax.experimental.pallas import tpu as pltpu

N_DEV = 8


def kernel(x, Win0, Wout0, Win1, Wout1, Win2, Wout2):
    m, d_loc = x.shape
    h_dim = Win0.shape[1]

    xb = x.astype(jnp.bfloat16)
    wins = [w.astype(jnp.bfloat16) for w in (Win0, Win1, Win2)]
    wouts = [w.astype(jnp.bfloat16) for w in (Wout0, Wout1, Wout2)]

    def body(x_ref, win0, win1, win2, wout0, wout1, wout2, out_ref,
             buf, hsum, xloc, send_sems, recv_sems):
        my = lax.axis_index("i")
        left = lax.rem(my + N_DEV - 1, N_DEV)
        right = lax.rem(my + 1, N_DEV)

        barrier = pltpu.get_barrier_semaphore()
        for nbr in (left, right):
            pl.semaphore_signal(barrier, inc=1, device_id=(nbr,),
                                device_id_type=pl.DeviceIdType.MESH)
        pl.semaphore_wait(barrier, 2)

        xloc[...] = x_ref[...]
        layers = ((win0, wout0), (win1, wout1), (win2, wout2))
        for l, (win, wout) in enumerate(layers):
            part = jnp.dot(xloc[...], win[...],
                           preferred_element_type=jnp.float32)
            hsum[...] = part
            buf[0, :, :] = part.astype(jnp.bfloat16)
            for h in range(N_DEV - 1):
                rdma = pltpu.make_async_remote_copy(
                    src_ref=buf.at[h],
                    dst_ref=buf.at[h + 1],
                    send_sem=send_sems.at[l, h],
                    recv_sem=recv_sems.at[l, h],
                    device_id=(right,),
                    device_id_type=pl.DeviceIdType.MESH,
                )
                rdma.start()
                rdma.wait()
                hsum[...] = hsum[...] + buf[h + 1].astype(jnp.float32)
            hrelu = jnp.maximum(hsum[...], 0.0).astype(jnp.bfloat16)
            res = jnp.dot(hrelu, wout[...],
                          preferred_element_type=jnp.float32)
            if l == len(layers) - 1:
                out_ref[...] = res
            else:
                xloc[...] = res.astype(jnp.bfloat16)

    return pl.pallas_call(
        body,
        out_shape=jax.ShapeDtypeStruct((m, d_loc), jnp.float32),
        in_specs=[pl.BlockSpec(memory_space=pltpu.VMEM)] * 7,
        out_specs=pl.BlockSpec(memory_space=pltpu.VMEM),
        scratch_shapes=[
            pltpu.VMEM((N_DEV, m, h_dim), jnp.bfloat16),
            pltpu.VMEM((m, h_dim), jnp.float32),
            pltpu.VMEM((m, d_loc), jnp.bfloat16),
            pltpu.SemaphoreType.DMA((3, N_DEV - 1)),
            pltpu.SemaphoreType.DMA((3, N_DEV - 1)),
        ],
        compiler_params=pltpu.CompilerParams(collective_id=0),
    )(xb, *wins, *wouts)


# device time: 54573 ns/iter; 2.3597x vs baseline; 2.3597x over previous
import jax
import jax.numpy as jnp
from jax import lax
from jax.experimental import pallas as pl
from jax.experimental.pallas import tpu as pltpu

N_DEV = 8


def kernel(x, Win0, Wout0, Win1, Wout1, Win2, Wout2):
    m, d_loc = x.shape
    h_dim = Win0.shape[1]
    chunk = h_dim // N_DEV

    xb = x.astype(jnp.bfloat16)
    wins = [w.astype(jnp.bfloat16) for w in (Win0, Win1, Win2)]
    wouts = [w.astype(jnp.bfloat16) for w in (Wout0, Wout1, Wout2)]

    def body(x_ref, win0, win1, win2, wout0, wout1, wout2, out_ref,
             pbuf, rs_buf, ag_buf, agchunk, hfull, xloc,
             send_sems, rs_sems, ag_sems):
        my = lax.axis_index("i")

        barrier = pltpu.get_barrier_semaphore()
        for t in range(N_DEV):
            @pl.when(t != my)
            def _():
                pl.semaphore_signal(barrier, inc=1, device_id=(t,),
                                    device_id_type=pl.DeviceIdType.MESH)
        pl.semaphore_wait(barrier, N_DEV - 1)

        xloc[...] = x_ref[...]
        layers = ((win0, wout0), (win1, wout1), (win2, wout2))
        for l, (win, wout) in enumerate(layers):
            part = jnp.dot(xloc[...], win[...],
                           preferred_element_type=jnp.float32)
            for t in range(N_DEV):
                pbuf[t, :, :] = part[:, t * chunk:(t + 1) * chunk].astype(
                    jnp.bfloat16)

            rs_rdmas = []
            for t in range(N_DEV):
                rdma = pltpu.make_async_remote_copy(
                    src_ref=pbuf.at[t],
                    dst_ref=rs_buf.at[my],
                    send_sem=send_sems.at[0, t],
                    recv_sem=rs_sems.at[my],
                    device_id=(t,),
                    device_id_type=pl.DeviceIdType.MESH,
                )
                rs_rdmas.append(rdma)

                @pl.when(t != my)
                def _(rdma=rdma):
                    rdma.start()

            for s in range(N_DEV):
                @pl.when(s != my)
                def _(s=s):
                    pltpu.make_async_remote_copy(
                        src_ref=pbuf.at[s],
                        dst_ref=rs_buf.at[s],
                        send_sem=send_sems.at[0, s],
                        recv_sem=rs_sems.at[s],
                        device_id=(s,),
                        device_id_type=pl.DeviceIdType.MESH,
                    ).wait_recv()

            rs_buf[my, :, :] = pbuf[my, :, :]
            csum = rs_buf[0].astype(jnp.float32)
            for s in range(1, N_DEV):
                csum = csum + rs_buf[s].astype(jnp.float32)
            agchunk[...] = jnp.maximum(csum, 0.0).astype(jnp.bfloat16)

            ag_rdmas = []
            for t in range(N_DEV):
                rdma = pltpu.make_async_remote_copy(
                    src_ref=agchunk,
                    dst_ref=ag_buf.at[my],
                    send_sem=send_sems.at[1, t],
                    recv_sem=ag_sems.at[my],
                    device_id=(t,),
                    device_id_type=pl.DeviceIdType.MESH,
                )
                ag_rdmas.append(rdma)

                @pl.when(t != my)
                def _(rdma=rdma):
                    rdma.start()

            for t in range(N_DEV):
                @pl.when(t != my)
                def _(rdma=rs_rdmas[t]):
                    rdma.wait_send()

            ag_buf[my, :, :] = agchunk[...]
            for s in range(N_DEV):
                @pl.when(s != my)
                def _(s=s):
                    pltpu.make_async_remote_copy(
                        src_ref=agchunk,
                        dst_ref=ag_buf.at[s],
                        send_sem=send_sems.at[1, s],
                        recv_sem=ag_sems.at[s],
                        device_id=(s,),
                        device_id_type=pl.DeviceIdType.MESH,
                    ).wait_recv()

            for s in range(N_DEV):
                hfull[:, s * chunk:(s + 1) * chunk] = ag_buf[s, :, :]

            for t in range(N_DEV):
                @pl.when(t != my)
                def _(rdma=ag_rdmas[t]):
                    rdma.wait_send()

            res = jnp.dot(hfull[...], wout[...],
                          preferred_element_type=jnp.float32)
            if l == len(layers) - 1:
                out_ref[...] = res
            else:
                xloc[...] = res.astype(jnp.bfloat16)

    return pl.pallas_call(
        body,
        out_shape=jax.ShapeDtypeStruct((m, d_loc), jnp.float32),
        in_specs=[pl.BlockSpec(memory_space=pltpu.VMEM)] * 7,
        out_specs=pl.BlockSpec(memory_space=pltpu.VMEM),
        scratch_shapes=[
            pltpu.VMEM((N_DEV, m, chunk), jnp.bfloat16),
            pltpu.VMEM((N_DEV, m, chunk), jnp.bfloat16),
            pltpu.VMEM((N_DEV, m, chunk), jnp.bfloat16),
            pltpu.VMEM((m, chunk), jnp.bfloat16),
            pltpu.VMEM((m, h_dim), jnp.bfloat16),
            pltpu.VMEM((m, d_loc), jnp.bfloat16),
            pltpu.SemaphoreType.DMA((2, N_DEV)),
            pltpu.SemaphoreType.DMA((N_DEV,)),
            pltpu.SemaphoreType.DMA((N_DEV,)),
        ],
        compiler_params=pltpu.CompilerParams(collective_id=0),
    )(xb, *wins, *wouts)


# device time: 50673 ns/iter; 2.5413x vs baseline; 1.0770x over previous
import jax
import jax.numpy as jnp
from jax import lax
from jax.experimental import pallas as pl
from jax.experimental.pallas import tpu as pltpu

N_DEV = 8


def kernel(x, Win0, Wout0, Win1, Wout1, Win2, Wout2):
    m, d_loc = x.shape
    h_dim = Win0.shape[1]
    chunk = h_dim // N_DEV


    def body(x_ref, win0, win1, win2, wout0, wout1, wout2, out_ref,
             pbuf, rs_buf, ag_buf, agchunk, hfull, xloc,
             send_sems, rs_sems, ag_sems):
        my = lax.axis_index("i")

        barrier = pltpu.get_barrier_semaphore()
        for t in range(N_DEV):
            @pl.when(t != my)
            def _():
                pl.semaphore_signal(barrier, inc=1, device_id=(t,),
                                    device_id_type=pl.DeviceIdType.MESH)
        pl.semaphore_wait(barrier, N_DEV - 1)

        xloc[...] = x_ref[...]
        layers = ((win0, wout0), (win1, wout1), (win2, wout2))
        for l, (win, wout) in enumerate(layers):
            rs_rdmas = []
            for t in range(N_DEV):
                pt = jnp.dot(xloc[...], win[:, t * chunk:(t + 1) * chunk],
                             preferred_element_type=jnp.float32)
                pbuf[t, :, :] = pt.astype(jnp.bfloat16)
                rdma = pltpu.make_async_remote_copy(
                    src_ref=pbuf.at[t],
                    dst_ref=rs_buf.at[my],
                    send_sem=send_sems.at[0, t],
                    recv_sem=rs_sems.at[my],
                    device_id=(t,),
                    device_id_type=pl.DeviceIdType.MESH,
                )
                rs_rdmas.append(rdma)

                @pl.when(t != my)
                def _(rdma=rdma):
                    rdma.start()

            for s in range(N_DEV):
                @pl.when(s != my)
                def _(s=s):
                    pltpu.make_async_remote_copy(
                        src_ref=pbuf.at[s],
                        dst_ref=rs_buf.at[s],
                        send_sem=send_sems.at[0, s],
                        recv_sem=rs_sems.at[s],
                        device_id=(s,),
                        device_id_type=pl.DeviceIdType.MESH,
                    ).wait_recv()

            rs_buf[my, :, :] = pbuf[my, :, :]
            csum = rs_buf[0].astype(jnp.float32)
            for s in range(1, N_DEV):
                csum = csum + rs_buf[s].astype(jnp.float32)
            agchunk[...] = jnp.maximum(csum, 0.0).astype(jnp.bfloat16)

            ag_rdmas = []
            for t in range(N_DEV):
                rdma = pltpu.make_async_remote_copy(
                    src_ref=agchunk,
                    dst_ref=ag_buf.at[my],
                    send_sem=send_sems.at[1, t],
                    recv_sem=ag_sems.at[my],
                    device_id=(t,),
                    device_id_type=pl.DeviceIdType.MESH,
                )
                ag_rdmas.append(rdma)

                @pl.when(t != my)
                def _(rdma=rdma):
                    rdma.start()

            for t in range(N_DEV):
                @pl.when(t != my)
                def _(rdma=rs_rdmas[t]):
                    rdma.wait_send()

            ag_buf[my, :, :] = agchunk[...]
            for s in range(N_DEV):
                @pl.when(s != my)
                def _(s=s):
                    pltpu.make_async_remote_copy(
                        src_ref=agchunk,
                        dst_ref=ag_buf.at[s],
                        send_sem=send_sems.at[1, s],
                        recv_sem=ag_sems.at[s],
                        device_id=(s,),
                        device_id_type=pl.DeviceIdType.MESH,
                    ).wait_recv()

            for s in range(N_DEV):
                hfull[:, s * chunk:(s + 1) * chunk] = ag_buf[s, :, :]

            for t in range(N_DEV):
                @pl.when(t != my)
                def _(rdma=ag_rdmas[t]):
                    rdma.wait_send()

            res = jnp.dot(hfull[...].astype(jnp.float32), wout[...],
                          preferred_element_type=jnp.float32)
            if l == len(layers) - 1:
                out_ref[...] = res
            else:
                xloc[...] = res

    return pl.pallas_call(
        body,
        out_shape=jax.ShapeDtypeStruct((m, d_loc), jnp.float32),
        in_specs=[pl.BlockSpec(memory_space=pltpu.VMEM)] * 7,
        out_specs=pl.BlockSpec(memory_space=pltpu.VMEM),
        scratch_shapes=[
            pltpu.VMEM((N_DEV, m, chunk), jnp.bfloat16),
            pltpu.VMEM((N_DEV, m, chunk), jnp.bfloat16),
            pltpu.VMEM((N_DEV, m, chunk), jnp.bfloat16),
            pltpu.VMEM((m, chunk), jnp.bfloat16),
            pltpu.VMEM((m, h_dim), jnp.bfloat16),
            pltpu.VMEM((m, d_loc), jnp.float32),
            pltpu.SemaphoreType.DMA((2, N_DEV)),
            pltpu.SemaphoreType.DMA((N_DEV,)),
            pltpu.SemaphoreType.DMA((N_DEV,)),
        ],
        compiler_params=pltpu.CompilerParams(
            collective_id=0,
            vmem_limit_bytes=60 * 1024 * 1024,
        ),
    )(x, Win0, Win1, Win2, Wout0, Wout1, Wout2)


# device time: 40078 ns/iter; 3.2131x vs baseline; 1.2644x over previous
import jax
import jax.numpy as jnp
from jax import lax
from jax.experimental import pallas as pl
from jax.experimental.pallas import tpu as pltpu

N_DEV = 8


def kernel(x, Win0, Wout0, Win1, Wout1, Win2, Wout2):
    m, d_loc = x.shape
    h_dim = Win0.shape[1]
    chunk = h_dim // N_DEV

    def body(x_ref, win0, win1, win2, wout0, wout1, wout2, out_ref,
             wv, wo, pbuf, rs_buf, ag_buf, agchunk, hfull, xloc,
             send_sems, rs_sems, ag_sems, wsems):
        my = lax.axis_index("i")
        wins_hbm = (win0, win1, win2)
        wouts_hbm = (wout0, wout1, wout2)

        def w_copies(l):
            return (
                pltpu.make_async_copy(wins_hbm[l], wv.at[l % 2],
                                      wsems.at[l, 0]),
                pltpu.make_async_copy(wouts_hbm[l], wo.at[l % 2],
                                      wsems.at[l, 1]),
            )

        for l in (0, 1):
            for c in w_copies(l):
                c.start()

        barrier = pltpu.get_barrier_semaphore()
        for t in range(N_DEV):
            @pl.when(t != my)
            def _():
                pl.semaphore_signal(barrier, inc=1, device_id=(t,),
                                    device_id_type=pl.DeviceIdType.MESH)
        pl.semaphore_wait(barrier, N_DEV - 1)

        xloc[...] = x_ref[...]
        for l in range(3):
            for c in w_copies(l):
                c.wait()
            win = wv.at[l % 2]
            wout = wo.at[l % 2]

            rs_rdmas = []
            for t in range(N_DEV):
                pt = jnp.dot(xloc[...], win[:, t * chunk:(t + 1) * chunk],
                             preferred_element_type=jnp.float32)
                pbuf[t, :, :] = pt.astype(jnp.bfloat16)
                rdma = pltpu.make_async_remote_copy(
                    src_ref=pbuf.at[t],
                    dst_ref=rs_buf.at[my],
                    send_sem=send_sems.at[0, t],
                    recv_sem=rs_sems.at[my],
                    device_id=(t,),
                    device_id_type=pl.DeviceIdType.MESH,
                )
                rs_rdmas.append(rdma)

                @pl.when(t != my)
                def _(rdma=rdma):
                    rdma.start()

            for s in range(N_DEV):
                @pl.when(s != my)
                def _(s=s):
                    pltpu.make_async_remote_copy(
                        src_ref=pbuf.at[s],
                        dst_ref=rs_buf.at[s],
                        send_sem=send_sems.at[0, s],
                        recv_sem=rs_sems.at[s],
                        device_id=(s,),
                        device_id_type=pl.DeviceIdType.MESH,
                    ).wait_recv()

            rs_buf[my, :, :] = pbuf[my, :, :]
            csum = rs_buf[0].astype(jnp.float32)
            for s in range(1, N_DEV):
                csum = csum + rs_buf[s].astype(jnp.float32)
            agchunk[...] = jnp.maximum(csum, 0.0).astype(jnp.bfloat16)

            ag_rdmas = []
            for t in range(N_DEV):
                rdma = pltpu.make_async_remote_copy(
                    src_ref=agchunk,
                    dst_ref=ag_buf.at[my],
                    send_sem=send_sems.at[1, t],
                    recv_sem=ag_sems.at[my],
                    device_id=(t,),
                    device_id_type=pl.DeviceIdType.MESH,
                )
                ag_rdmas.append(rdma)

                @pl.when(t != my)
                def _(rdma=rdma):
                    rdma.start()

            for t in range(N_DEV):
                @pl.when(t != my)
                def _(rdma=rs_rdmas[t]):
                    rdma.wait_send()

            ag_buf[my, :, :] = agchunk[...]
            for s in range(N_DEV):
                @pl.when(s != my)
                def _(s=s):
                    pltpu.make_async_remote_copy(
                        src_ref=agchunk,
                        dst_ref=ag_buf.at[s],
                        send_sem=send_sems.at[1, s],
                        recv_sem=ag_sems.at[s],
                        device_id=(s,),
                        device_id_type=pl.DeviceIdType.MESH,
                    ).wait_recv()

            for s in range(N_DEV):
                hfull[:, s * chunk:(s + 1) * chunk] = ag_buf[s, :, :]

            for t in range(N_DEV):
                @pl.when(t != my)
                def _(rdma=ag_rdmas[t]):
                    rdma.wait_send()

            res = jnp.dot(hfull[...].astype(jnp.float32), wout[...],
                          preferred_element_type=jnp.float32)
            if l == 2:
                out_ref[...] = res
            else:
                xloc[...] = res
                if l == 0:
                    for c in w_copies(2):
                        c.start()

    return pl.pallas_call(
        body,
        out_shape=jax.ShapeDtypeStruct((m, d_loc), jnp.float32),
        in_specs=[pl.BlockSpec(memory_space=pltpu.VMEM)]
        + [pl.BlockSpec(memory_space=pl.ANY)] * 6,
        out_specs=pl.BlockSpec(memory_space=pltpu.VMEM),
        scratch_shapes=[
            pltpu.VMEM((2, d_loc, h_dim), jnp.float32),
            pltpu.VMEM((2, h_dim, d_loc), jnp.float32),
            pltpu.VMEM((N_DEV, m, chunk), jnp.bfloat16),
            pltpu.VMEM((N_DEV, m, chunk), jnp.bfloat16),
            pltpu.VMEM((N_DEV, m, chunk), jnp.bfloat16),
            pltpu.VMEM((m, chunk), jnp.bfloat16),
            pltpu.VMEM((m, h_dim), jnp.bfloat16),
            pltpu.VMEM((m, d_loc), jnp.float32),
            pltpu.SemaphoreType.DMA((2, N_DEV)),
            pltpu.SemaphoreType.DMA((N_DEV,)),
            pltpu.SemaphoreType.DMA((N_DEV,)),
            pltpu.SemaphoreType.DMA((3, 2)),
        ],
        compiler_params=pltpu.CompilerParams(
            collective_id=0,
            vmem_limit_bytes=60 * 1024 * 1024,
        ),
    )(x, Win0, Win1, Win2, Wout0, Wout1, Wout2)


# device time: 38543 ns/iter; 3.3410x vs baseline; 1.0398x over previous
import jax
import jax.numpy as jnp
from jax import lax
from jax.experimental import pallas as pl
from jax.experimental.pallas import tpu as pltpu

N_DEV = 8


def kernel(x, Win0, Wout0, Win1, Wout1, Win2, Wout2):
    m, d_loc = x.shape
    h_dim = Win0.shape[1]
    chunk = h_dim // N_DEV

    def body(x_ref, win0, win1, win2, wout0, wout1, wout2, out_ref,
             wv, wo, winb, woutb, pbuf, rs_buf, ag_buf, agchunk, xloc,
             send_sems, rs_sems, ag_sems, wsems):
        my = lax.axis_index("i")
        wins_hbm = (win0, win1, win2)
        wouts_hbm = (wout0, wout1, wout2)

        def w_copies(l):
            return (
                pltpu.make_async_copy(wins_hbm[l], wv.at[l % 2],
                                      wsems.at[l, 0]),
                pltpu.make_async_copy(wouts_hbm[l], wo.at[l % 2],
                                      wsems.at[l, 1]),
            )

        for l in (0, 1):
            for c in w_copies(l):
                c.start()

        barrier = pltpu.get_barrier_semaphore()
        for t in range(N_DEV):
            @pl.when(t != my)
            def _():
                pl.semaphore_signal(barrier, inc=1, device_id=(t,),
                                    device_id_type=pl.DeviceIdType.MESH)

        xloc[...] = x_ref[...]
        for l in range(3):
            sl = l % 2
            if l == 0:
                for c in w_copies(0):
                    c.wait()

            if l == 0:
                for t in range(N_DEV):
                    pt = jnp.dot(xloc[...],
                                 wv.at[sl][:, t * chunk:(t + 1) * chunk],
                                 preferred_element_type=jnp.float32)
                    pbuf[t, :, :] = pt.astype(jnp.bfloat16)
                pl.semaphore_wait(barrier, N_DEV - 1)
            else:
                xb = xloc[...].astype(jnp.bfloat16)

            rs_rdmas = []
            for t in range(N_DEV):
                if l > 0:
                    pt = jnp.dot(xb, winb.at[sl][:, t * chunk:(t + 1) * chunk],
                                 preferred_element_type=jnp.float32)
                    pbuf[t, :, :] = pt.astype(jnp.bfloat16)
                rdma = pltpu.make_async_remote_copy(
                    src_ref=pbuf.at[t],
                    dst_ref=rs_buf.at[my],
                    send_sem=send_sems.at[0, t],
                    recv_sem=rs_sems.at[my],
                    device_id=(t,),
                    device_id_type=pl.DeviceIdType.MESH,
                )
                rs_rdmas.append(rdma)

                @pl.when(t != my)
                def _(rdma=rdma):
                    rdma.start()

            for s in range(N_DEV):
                @pl.when(s != my)
                def _(s=s):
                    pltpu.make_async_remote_copy(
                        src_ref=pbuf.at[s],
                        dst_ref=rs_buf.at[s],
                        send_sem=send_sems.at[0, s],
                        recv_sem=rs_sems.at[s],
                        device_id=(s,),
                        device_id_type=pl.DeviceIdType.MESH,
                    ).wait_recv()

            rs_buf[my, :, :] = pbuf[my, :, :]
            csum = rs_buf[0].astype(jnp.float32)
            for s in range(1, N_DEV):
                csum = csum + rs_buf[s].astype(jnp.float32)
            agchunk[...] = jnp.maximum(csum, 0.0).astype(jnp.bfloat16)

            ag_rdmas = []
            for t in range(N_DEV):
                rdma = pltpu.make_async_remote_copy(
                    src_ref=agchunk,
                    dst_ref=ag_buf.at[my],
                    send_sem=send_sems.at[1, t],
                    recv_sem=ag_sems.at[my],
                    device_id=(t,),
                    device_id_type=pl.DeviceIdType.MESH,
                )
                ag_rdmas.append(rdma)

                @pl.when(t != my)
                def _(rdma=rdma):
                    rdma.start()

            for t in range(N_DEV):
                @pl.when(t != my)
                def _(rdma=rs_rdmas[t]):
                    rdma.wait_send()

            ag_buf[my, :, :] = agchunk[...]

            if l < 2:
                for c in w_copies(l + 1):
                    c.wait()
                nsl = (l + 1) % 2
                winb[nsl, :, :] = wv[nsl].astype(jnp.bfloat16)
                woutb[nsl, :, :] = wo[nsl].astype(jnp.bfloat16)

            acc_ref = out_ref if l == 2 else xloc
            wout_l = wo.at[sl] if l == 0 else woutb.at[sl]
            for s in range(N_DEV):
                @pl.when(s != my)
                def _(s=s):
                    pltpu.make_async_remote_copy(
                        src_ref=agchunk,
                        dst_ref=ag_buf.at[s],
                        send_sem=send_sems.at[1, s],
                        recv_sem=ag_sems.at[s],
                        device_id=(s,),
                        device_id_type=pl.DeviceIdType.MESH,
                    ).wait_recv()
                lhs = ag_buf[s]
                if l == 0:
                    lhs = lhs.astype(jnp.float32)
                contrib = jnp.dot(lhs,
                                  wout_l[s * chunk:(s + 1) * chunk, :],
                                  preferred_element_type=jnp.float32)
                if s == 0:
                    acc_ref[...] = contrib
                else:
                    acc_ref[...] = acc_ref[...] + contrib

            for t in range(N_DEV):
                @pl.when(t != my)
                def _(rdma=ag_rdmas[t]):
                    rdma.wait_send()

            if l == 0:
                for c in w_copies(2):
                    c.start()

    return pl.pallas_call(
        body,
        out_shape=jax.ShapeDtypeStruct((m, d_loc), jnp.float32),
        in_specs=[pl.BlockSpec(memory_space=pltpu.VMEM)]
        + [pl.BlockSpec(memory_space=pl.ANY)] * 6,
        out_specs=pl.BlockSpec(memory_space=pltpu.VMEM),
        scratch_shapes=[
            pltpu.VMEM((2, d_loc, h_dim), jnp.float32),
            pltpu.VMEM((2, h_dim, d_loc), jnp.float32),
            pltpu.VMEM((2, d_loc, h_dim), jnp.bfloat16),
            pltpu.VMEM((2, h_dim, d_loc), jnp.bfloat16),
            pltpu.VMEM((N_DEV, m, chunk), jnp.bfloat16),
            pltpu.VMEM((N_DEV, m, chunk), jnp.bfloat16),
            pltpu.VMEM((N_DEV, m, chunk), jnp.bfloat16),
            pltpu.VMEM((m, chunk), jnp.bfloat16),
            pltpu.VMEM((m, d_loc), jnp.float32),
            pltpu.SemaphoreType.DMA((2, N_DEV)),
            pltpu.SemaphoreType.DMA((N_DEV,)),
            pltpu.SemaphoreType.DMA((N_DEV,)),
            pltpu.SemaphoreType.DMA((3, 2)),
        ],
        compiler_params=pltpu.CompilerParams(
            collective_id=0,
            vmem_limit_bytes=60 * 1024 * 1024,
        ),
    )(x, Win0, Win1, Win2, Wout0, Wout1, Wout2)


# device time: 36670 ns/iter; 3.5117x vs baseline; 1.0511x over previous
import jax
import jax.numpy as jnp
from jax import lax
from jax.experimental import pallas as pl
from jax.experimental.pallas import tpu as pltpu

N_DEV = 8


def kernel(x, Win0, Wout0, Win1, Wout1, Win2, Wout2):
    m, d_loc = x.shape
    h_dim = Win0.shape[1]
    chunk = h_dim // N_DEV

    def body(x_ref, win0, win1, win2, wout0, wout1, wout2, out_ref,
             wv, wo, winb, woutb, pbuf, rs_buf, ag_buf, agchunk, xloc,
             send_sems, rs_sems, ag_sems, wsems):
        my = lax.axis_index("i")
        wins_hbm = (win0, win1, win2)
        wouts_hbm = (wout0, wout1, wout2)

        def w_copies(l):
            return (
                pltpu.make_async_copy(wins_hbm[l], wv.at[l % 2],
                                      wsems.at[l, 0]),
                pltpu.make_async_copy(wouts_hbm[l], wo.at[l % 2],
                                      wsems.at[l, 1]),
            )

        for l in (0, 1):
            for c in w_copies(l):
                c.start()

        barrier = pltpu.get_barrier_semaphore()
        for t in range(N_DEV):
            @pl.when(t != my)
            def _():
                pl.semaphore_signal(barrier, inc=1, device_id=(t,),
                                    device_id_type=pl.DeviceIdType.MESH)

        xloc[...] = x_ref[...]
        for l in range(3):
            sl = l % 2
            if l == 0:
                w_copies(0)[0].wait()

            if l == 0:
                for t in range(N_DEV):
                    pt = jnp.dot(xloc[...],
                                 wv.at[sl][:, t * chunk:(t + 1) * chunk],
                                 preferred_element_type=jnp.float32)
                    pbuf[t, :, :] = pt.astype(jnp.bfloat16)
                pl.semaphore_wait(barrier, N_DEV - 1)
            else:
                xb = xloc[...].astype(jnp.bfloat16)

            rs_rdmas = []
            for t in range(N_DEV):
                if l > 0:
                    pt = jnp.dot(xb, winb.at[sl][:, t * chunk:(t + 1) * chunk],
                                 preferred_element_type=jnp.float32)
                    pbuf[t, :, :] = pt.astype(jnp.bfloat16)
                rdma = pltpu.make_async_remote_copy(
                    src_ref=pbuf.at[t],
                    dst_ref=rs_buf.at[my],
                    send_sem=send_sems.at[0, t],
                    recv_sem=rs_sems.at[my],
                    device_id=(t,),
                    device_id_type=pl.DeviceIdType.MESH,
                )
                rs_rdmas.append(rdma)

                @pl.when(t != my)
                def _(rdma=rdma):
                    rdma.start()

            for s in range(N_DEV):
                @pl.when(s != my)
                def _(s=s):
                    pltpu.make_async_remote_copy(
                        src_ref=pbuf.at[s],
                        dst_ref=rs_buf.at[s],
                        send_sem=send_sems.at[0, s],
                        recv_sem=rs_sems.at[s],
                        device_id=(s,),
                        device_id_type=pl.DeviceIdType.MESH,
                    ).wait_recv()

            rs_buf[my, :, :] = pbuf[my, :, :]
            csum = rs_buf[0].astype(jnp.float32)
            for s in range(1, N_DEV):
                csum = csum + rs_buf[s].astype(jnp.float32)
            agchunk[...] = jnp.maximum(csum, 0.0).astype(jnp.bfloat16)

            ag_rdmas = []
            for t in range(N_DEV):
                rdma = pltpu.make_async_remote_copy(
                    src_ref=agchunk,
                    dst_ref=ag_buf.at[my],
                    send_sem=send_sems.at[1, t],
                    recv_sem=ag_sems.at[my],
                    device_id=(t,),
                    device_id_type=pl.DeviceIdType.MESH,
                )
                ag_rdmas.append(rdma)

                @pl.when(t != my)
                def _(rdma=rdma):
                    rdma.start()

            for t in range(N_DEV):
                @pl.when(t != my)
                def _(rdma=rs_rdmas[t]):
                    rdma.wait_send()

            ag_buf[my, :, :] = agchunk[...]

            if l < 2:
                for c in w_copies(l + 1):
                    c.wait()
                nsl = (l + 1) % 2
                winb[nsl, :, :] = wv[nsl].astype(jnp.bfloat16)
                woutb[nsl, :, :] = wo[nsl].astype(jnp.bfloat16)

            acc_ref = out_ref if l == 2 else xloc
            if l == 0:
                w_copies(0)[1].wait()
            wout_l = wo.at[sl] if l == 0 else woutb.at[sl]
            for s in range(N_DEV):
                @pl.when(s != my)
                def _(s=s):
                    pltpu.make_async_remote_copy(
                        src_ref=agchunk,
                        dst_ref=ag_buf.at[s],
                        send_sem=send_sems.at[1, s],
                        recv_sem=ag_sems.at[s],
                        device_id=(s,),
                        device_id_type=pl.DeviceIdType.MESH,
                    ).wait_recv()
                lhs = ag_buf[s]
                if l == 0:
                    lhs = lhs.astype(jnp.float32)
                contrib = jnp.dot(lhs,
                                  wout_l[s * chunk:(s + 1) * chunk, :],
                                  preferred_element_type=jnp.float32)
                if s == 0:
                    acc_ref[...] = contrib
                else:
                    acc_ref[...] = acc_ref[...] + contrib

            for t in range(N_DEV):
                @pl.when(t != my)
                def _(rdma=ag_rdmas[t]):
                    rdma.wait_send()

            if l == 0:
                for c in w_copies(2):
                    c.start()

    return pl.pallas_call(
        body,
        out_shape=jax.ShapeDtypeStruct((m, d_loc), jnp.float32),
        in_specs=[pl.BlockSpec(memory_space=pltpu.VMEM)]
        + [pl.BlockSpec(memory_space=pl.ANY)] * 6,
        out_specs=pl.BlockSpec(memory_space=pltpu.VMEM),
        scratch_shapes=[
            pltpu.VMEM((2, d_loc, h_dim), jnp.float32),
            pltpu.VMEM((2, h_dim, d_loc), jnp.float32),
            pltpu.VMEM((2, d_loc, h_dim), jnp.bfloat16),
            pltpu.VMEM((2, h_dim, d_loc), jnp.bfloat16),
            pltpu.VMEM((N_DEV, m, chunk), jnp.bfloat16),
            pltpu.VMEM((N_DEV, m, chunk), jnp.bfloat16),
            pltpu.VMEM((N_DEV, m, chunk), jnp.bfloat16),
            pltpu.VMEM((m, chunk), jnp.bfloat16),
            pltpu.VMEM((m, d_loc), jnp.float32),
            pltpu.SemaphoreType.DMA((2, N_DEV)),
            pltpu.SemaphoreType.DMA((N_DEV,)),
            pltpu.SemaphoreType.DMA((N_DEV,)),
            pltpu.SemaphoreType.DMA((3, 2)),
        ],
        compiler_params=pltpu.CompilerParams(
            collective_id=0,
            vmem_limit_bytes=60 * 1024 * 1024,
        ),
    )(x, Win0, Win1, Win2, Wout0, Wout1, Wout2)
